# baseline (speedup 1.0000x reference)
import functools

import jax
import jax.numpy as jnp
from jax import lax
from jax.experimental import pallas as pl
from jax.experimental.pallas import tpu as pltpu
from jax.experimental.pallas import tpu_sc as plsc

B = 16384
C = 1000
NC = 2
NS = 16
L = 16
NW = NC * NS
CPW = B // NW
SUB = 8
NV = CPW // L

_LN2 = 0.6931471805599453
_SQRT2 = 1.4142135623730951


def _log_f32(x):
    bits = lax.bitcast_convert_type(x, jnp.int32)
    e = (bits >> 23) - 127
    mbits = (bits & 0x007FFFFF) | 0x3F800000
    m = lax.bitcast_convert_type(mbits, jnp.float32)
    big = m > _SQRT2
    m = jnp.where(big, m * 0.5, m)
    e = jnp.where(big, e + 1, e)
    s = (m - 1.0) / (m + 1.0)
    z = s * s
    p = 1.0 / 9.0
    p = 1.0 / 7.0 + z * p
    p = 1.0 / 5.0 + z * p
    p = 1.0 / 3.0 + z * p
    p = 1.0 + z * p
    return e.astype(jnp.float32) * _LN2 + 2.0 * s * p


@functools.partial(
    pl.kernel,
    out_type=jax.ShapeDtypeStruct((B,), jnp.float32),
    mesh=plsc.VectorSubcoreMesh(core_axis_name="c", subcore_axis_name="s"),
    scratch_types=[
        pltpu.VMEM((L,), jnp.int32),
        pltpu.VMEM((SUB, CPW), jnp.float32),
        pltpu.VMEM((SUB, CPW), jnp.float32),
        pltpu.VMEM((SUB, CPW), jnp.float32),
        pltpu.VMEM((CPW,), jnp.float32),
        pltpu.SemaphoreType.DMA,
    ],
)
def _transition_loss_sc(at_hbm, bt_hbm, gt_hbm, cols_hbm, out_hbm,
                        cols_v, da, db, dg, out_v, sem):
    wid = lax.axis_index("s") * NC + lax.axis_index("c")
    base = wid * CPW

    pltpu.sync_copy(cols_hbm, cols_v)
    cv = cols_v[pl.ds(0, L)]

    rows = []
    copies = []
    for t, (src, dst) in enumerate(((at_hbm, da), (bt_hbm, db),
                                    (gt_hbm, dg))):
        c = cv[t]
        r0 = pl.multiple_of(lax.div(c, SUB) * SUB, SUB)
        rows.append(jnp.full((L,), lax.rem(c, SUB),
                             dtype=jnp.int32).astype(jnp.float32))
        copies.append(pltpu.async_copy(
            src.at[pl.ds(r0, SUB), pl.ds(base, CPW)], dst, sem))
    for cp in copies:
        cp.wait()

    def extract(ref, row_vec, sl):
        acc = jnp.zeros((L,), dtype=jnp.float32)
        for j in range(SUB):
            d = row_vec - float(j)
            m = jnp.maximum(1.0 - d * d, 0.0)
            acc = acc + m * ref[j, sl]
        return acc

    for i in range(NV):
        sl = pl.ds(i * L, L)
        a = extract(da, rows[0], sl)
        b = extract(db, rows[1], sl)
        g = extract(dg, rows[2], sl)
        x = jnp.maximum(1.0 - jnp.exp(g), 1e-8)
        val = a + b - _log_f32(x)
        out_v[sl] = jnp.maximum(val, 0.0)

    pltpu.sync_copy(out_v, out_hbm.at[pl.ds(base, CPW)])


def kernel(log_y_alpha, log_y_beta, log_y_gamma,
           alpha_index, beta_index, gamma_index):
    cols = jnp.stack([
        jnp.asarray(alpha_index, dtype=jnp.int32),
        jnp.asarray(beta_index, dtype=jnp.int32),
        jnp.asarray(gamma_index, dtype=jnp.int32),
    ])
    cols = jnp.pad(cols, (0, L - 3))
    return _transition_loss_sc(
        log_y_alpha.T, log_y_beta.T, log_y_gamma.T, cols)


# speedup vs baseline: 5.7681x; 5.7681x over previous
import jax
import jax.numpy as jnp
from jax import lax
from jax.experimental import pallas as pl
from jax.experimental.pallas import tpu as pltpu

B = 16384
C = 1000
SUB = 8


NCH = 2
CHB = B // NCH


def _body(c0_ref, c1_ref, c2_ref, a_any, b_any, g_any, out_ref,
          a_v, b_v, g_v, sems):
    crefs = (c2_ref, c0_ref, c1_ref)
    srcs_dsts = ((g_any, g_v), (a_any, a_v), (b_any, b_v))
    copies = []
    for k in range(NCH):
        csl = pl.ds(k * CHB, CHB)
        chunk_copies = []
        for t, (src, dst) in enumerate(srcs_dsts):
            r0 = pl.multiple_of(lax.div(crefs[t][0], SUB) * SUB, SUB)
            cp = pltpu.make_async_copy(
                src.at[pl.ds(r0, SUB), csl], dst.at[:, csl], sems.at[t, k])
            cp.start()
            chunk_copies.append(cp)
        copies.append(chunk_copies)

    sub_ids = lax.broadcasted_iota(jnp.int32, (SUB, CHB), 0)

    def sel(cref):
        row = lax.rem(cref[0], SUB)
        return (sub_ids == row).astype(jnp.float32)

    sel_a, sel_b, sel_g = sel(c0_ref), sel(c1_ref), sel(c2_ref)

    for k in range(NCH):
        csl = pl.ds(k * CHB, CHB)
        copies[k][0].wait()
        g = jnp.sum(sel_g * g_v[:, csl], axis=0)
        x = jnp.maximum(1.0 - jnp.exp(g), 1e-8)
        lg = jnp.log(x)
        copies[k][1].wait()
        copies[k][2].wait()
        ab = jnp.sum(sel_a * a_v[:, csl] + sel_b * b_v[:, csl], axis=0)
        out_ref[csl] = jnp.maximum(ab - lg, 0.0)


@jax.jit
def _transition_loss_tc(at, bt, gt, c0, c1, c2):
    return pl.pallas_call(
        _body,
        grid_spec=pltpu.PrefetchScalarGridSpec(
            num_scalar_prefetch=3,
            grid=(),
            in_specs=[pl.BlockSpec(memory_space=pl.ANY)] * 3,
            out_specs=pl.BlockSpec(memory_space=pltpu.VMEM),
            scratch_shapes=[
                pltpu.VMEM((SUB, B), jnp.float32),
                pltpu.VMEM((SUB, B), jnp.float32),
                pltpu.VMEM((SUB, B), jnp.float32),
                pltpu.SemaphoreType.DMA((3, NCH)),
            ],
        ),
        out_shape=jax.ShapeDtypeStruct((B,), jnp.float32),
    )(c0, c1, c2, at, bt, gt)


def kernel(log_y_alpha, log_y_beta, log_y_gamma,
           alpha_index, beta_index, gamma_index):
    c0 = jnp.asarray(alpha_index, dtype=jnp.int32).reshape(1)
    c1 = jnp.asarray(beta_index, dtype=jnp.int32).reshape(1)
    c2 = jnp.asarray(gamma_index, dtype=jnp.int32).reshape(1)
    return _transition_loss_tc(
        log_y_alpha.T, log_y_beta.T, log_y_gamma.T, c0, c1, c2)


# speedup vs baseline: 6.0828x; 1.0546x over previous
import jax
import jax.numpy as jnp
from jax import lax
from jax.experimental import pallas as pl
from jax.experimental.pallas import tpu as pltpu

B = 16384
C = 1000
SUB = 8


NCH = 2
CHB = B // NCH


def _body(c0_ref, c1_ref, c2_ref, a_any, b_any, g_any, out_ref,
          a_v, b_v, g_v, sems):
    crefs = (c2_ref, c0_ref, c1_ref)
    srcs_dsts = ((g_any, g_v), (a_any, a_v), (b_any, b_v))
    copies = []
    for k in range(NCH):
        csl = pl.ds(k * CHB, CHB)
        chunk_copies = []
        for t, (src, dst) in enumerate(srcs_dsts):
            r0 = pl.multiple_of(lax.div(crefs[t][0], SUB) * SUB, SUB)
            cp = pltpu.make_async_copy(
                src.at[pl.ds(r0, SUB), csl], dst.at[:, csl], sems.at[t, k])
            cp.start()
            chunk_copies.append(cp)
        copies.append(chunk_copies)

    def pick(ref, cref, csl):
        row = lax.rem(cref[0], SUB)
        return ref[pl.ds(row, 1), csl].reshape(CHB)

    for k in range(NCH):
        csl = pl.ds(k * CHB, CHB)
        copies[k][0].wait()
        g = pick(g_v, c2_ref, csl)
        x = jnp.maximum(1.0 - jnp.exp(g), 1e-8)
        lg = jnp.log(x)
        copies[k][1].wait()
        copies[k][2].wait()
        ab = pick(a_v, c0_ref, csl) + pick(b_v, c1_ref, csl)
        out_ref[csl] = jnp.maximum(ab - lg, 0.0)


@jax.jit
def _transition_loss_tc(at, bt, gt, c0, c1, c2):
    return pl.pallas_call(
        _body,
        grid_spec=pltpu.PrefetchScalarGridSpec(
            num_scalar_prefetch=3,
            grid=(),
            in_specs=[pl.BlockSpec(memory_space=pl.ANY)] * 3,
            out_specs=pl.BlockSpec(memory_space=pltpu.VMEM),
            scratch_shapes=[
                pltpu.VMEM((SUB, B), jnp.float32),
                pltpu.VMEM((SUB, B), jnp.float32),
                pltpu.VMEM((SUB, B), jnp.float32),
                pltpu.SemaphoreType.DMA((3, NCH)),
            ],
        ),
        out_shape=jax.ShapeDtypeStruct((B,), jnp.float32),
    )(c0, c1, c2, at, bt, gt)


def kernel(log_y_alpha, log_y_beta, log_y_gamma,
           alpha_index, beta_index, gamma_index):
    c0 = jnp.asarray(alpha_index, dtype=jnp.int32).reshape(1)
    c1 = jnp.asarray(beta_index, dtype=jnp.int32).reshape(1)
    c2 = jnp.asarray(gamma_index, dtype=jnp.int32).reshape(1)
    return _transition_loss_tc(
        log_y_alpha.T, log_y_beta.T, log_y_gamma.T, c0, c1, c2)
